# baseline (speedup 1.0000x reference)
import functools

import jax
import jax.numpy as jnp
from jax import lax
from jax.experimental import pallas as pl
from jax.experimental.pallas import tpu as pltpu
from jax.experimental.pallas import tpu_sc as plsc

D = 256
DW = D // 2
S = 16
EMB = 256
NC = 2
NS = 16
NW = NC * NS
BP = 10240
B_SC = 8192
B_TC = BP - B_SC
SEEDS_PER_W = B_SC // NW
CS = 8
NCHUNK = SEEDS_PER_W // CS
NBUF = 4
TB = 2048
PACK_ROWS = 2000
TCG = 32
TCROWS = TCG * (S + 1)


def _rne16(f):
    u = lax.bitcast_convert_type(f, jnp.int32)
    odd = lax.bitwise_and(
        lax.shift_right_logical(u, jnp.int32(16)), jnp.int32(1))
    r = lax.shift_right_logical(u + jnp.int32(32767) + odd, jnp.int32(16))
    return lax.bitwise_and(r, jnp.int32(65535))


def _lo_f32(x):
    return lax.bitcast_convert_type(lax.shift_left(x, jnp.int32(16)),
                                    jnp.float32)


def _hi_f32(x):
    return lax.bitcast_convert_type(lax.bitwise_and(x, jnp.int32(-65536)),
                                    jnp.float32)


def _pack_body(f_ref, o_ref):
    x = f_ref[...]
    lo = _rne16(x[:, :DW])
    hi = _rne16(x[:, DW:])
    o_ref[...] = lax.bitwise_or(lax.shift_left(hi, jnp.int32(16)), lo)


def _make_sc_gather_sum():
    mesh = plsc.VectorSubcoreMesh(core_axis_name="c", subcore_axis_name="s")

    @functools.partial(
        pl.kernel,
        mesh=mesh,
        out_type=jax.ShapeDtypeStruct((B_SC, 2 * DW), jnp.int32),
        scratch_types=(
            [pltpu.VMEM((SEEDS_PER_W * S,), jnp.int32),
             pltpu.VMEM((SEEDS_PER_W,), jnp.int32)]
            + [pltpu.VMEM((CS * S, DW), jnp.int32)] * NBUF
            + [pltpu.VMEM((CS, DW), jnp.int32)] * NBUF
            + [pltpu.VMEM((CS, 2 * DW), jnp.int32)] * NBUF
            + [pltpu.SemaphoreType.DMA] * (2 * NBUF)
        ),
    )
    def sc_gather_sum(feat_hbm, nodes_hbm, neigh_hbm, comb_out,
                      nidx_v, sidx_v, *bufs):
        nbufs = bufs[0:NBUF]
        sbufs = bufs[NBUF:2 * NBUF]
        obufs = bufs[2 * NBUF:3 * NBUF]
        gsems = bufs[3 * NBUF:4 * NBUF]
        osems = bufs[4 * NBUF:5 * NBUF]
        wid = lax.axis_index("s") * NC + lax.axis_index("c")
        base = pl.multiple_of(wid * SEEDS_PER_W, SEEDS_PER_W)
        pltpu.sync_copy(neigh_hbm.at[pl.ds(base * S, SEEDS_PER_W * S)], nidx_v)
        pltpu.sync_copy(nodes_hbm.at[pl.ds(base, SEEDS_PER_W)], sidx_v)

        def fire_gather(g, b):
            off_n = pl.multiple_of(g * (CS * S), CS * S)
            off_s = pl.multiple_of(g * CS, CS)
            pltpu.async_copy(
                feat_hbm.at[nidx_v.at[pl.ds(off_n, CS * S)]], nbufs[b], gsems[b])
            pltpu.async_copy(
                feat_hbm.at[sidx_v.at[pl.ds(off_s, CS)]], sbufs[b], gsems[b])

        def wait_gather(b):
            pltpu.make_async_copy(
                feat_hbm.at[pl.ds(0, CS * S)], nbufs[b], gsems[b]).wait()
            pltpu.make_async_copy(
                feat_hbm.at[pl.ds(0, CS)], sbufs[b], gsems[b]).wait()

        def fire_out(g, b):
            row = pl.multiple_of(base + g * CS, CS)
            pltpu.async_copy(obufs[b], comb_out.at[pl.ds(row, CS)], osems[b])

        def drain_out(b):
            pltpu.make_async_copy(
                obufs[b], comb_out.at[pl.ds(0, CS)], osems[b]).wait()

        def compute(b):
            nb, sb, ob = nbufs[b], sbufs[b], obufs[b]

            def seed_body(s0, _):
                r0 = s0 * S
                for v in range(DW // 16):
                    x = nb[r0, pl.ds(v * 16, 16)]
                    a_lo, a_hi = _lo_f32(x), _hi_f32(x)
                    for r in range(1, S):
                        y = nb[r0 + r, pl.ds(v * 16, 16)]
                        a_lo = a_lo + _lo_f32(y)
                        a_hi = a_hi + _hi_f32(y)
                    word = lax.bitwise_or(
                        lax.shift_left(_rne16(a_hi), jnp.int32(16)),
                        _rne16(a_lo))
                    ob[s0, pl.ds(DW + v * 16, 16)] = word
                    ob[s0, pl.ds(v * 16, 16)] = sb[s0, pl.ds(v * 16, 16)]
                return 0

            lax.fori_loop(0, CS, seed_body, 0, unroll=False)

        for b in range(NBUF):
            fire_gather(b, b)

        def ring_body(p, _):
            for b in range(NBUF):
                g = p * NBUF + b
                wait_gather(b)
                compute(b)

                @pl.when(p > 0)
                def _():
                    drain_out(b)

                fire_out(g, b)

                @pl.when(g + NBUF < NCHUNK)
                def _():
                    fire_gather(g + NBUF, b)
            return 0

        lax.fori_loop(0, NCHUNK // NBUF, ring_body, 0, unroll=False)
        for b in range(NBUF):
            drain_out(b)

    return sc_gather_sum


_sc_gather_sum = _make_sc_gather_sum()


def _tc_gather_body(idx_ref, feat_ref, comb_ref, rows0, rows1, sem0, sem1):
    i = pl.program_id(0)
    n = pl.num_programs(0)

    def fire(step, buf, sem):
        base = step * TCROWS
        for j in range(TCROWS):
            pltpu.make_async_copy(
                feat_ref.at[pl.ds(idx_ref[base + j], 1)],
                buf.at[pl.ds(j, 1)], sem).start()

    def drain(buf, sem):
        for j in range(TCROWS):
            pltpu.make_async_copy(
                feat_ref.at[pl.ds(0, 1)], buf.at[pl.ds(j, 1)], sem).wait()

    def consume(buf):
        x = buf[...]
        acc = x[0:TCG]
        for k in range(1, S):
            acc = acc + x[k * TCG:(k + 1) * TCG]
        sf = x[S * TCG:(S + 1) * TCG]
        comb_ref[...] = jnp.concatenate(
            [sf, acc * jnp.float32(1.0 / S)], axis=1)

    @pl.when(i == 0)
    def _():
        fire(0, rows0, sem0)

    @pl.when(i % 2 == 0)
    def _():
        @pl.when(i + 1 < n)
        def _():
            fire(i + 1, rows1, sem1)

        drain(rows0, sem0)
        consume(rows0)

    @pl.when(i % 2 == 1)
    def _():
        @pl.when(i + 1 < n)
        def _():
            fire(i + 1, rows0, sem0)

        drain(rows1, sem1)
        consume(rows1)


def _mm_f32_body(w_ref, c_ref, o_ref):
    w = w_ref[...]
    dn = (((1,), (1,)), ((), ()))
    acc = lax.dot_general(w[:, :D], c_ref[:, :D], dn,
                          preferred_element_type=jnp.float32)
    acc += lax.dot_general(w[:, D:], c_ref[:, D:], dn,
                           preferred_element_type=jnp.float32)
    o_ref[...] = jnp.maximum(acc, 0.0)


def _mm_body(w_ref, c_ref, o_ref):
    w = w_ref[...]
    cw = c_ref[...]
    sw = cw[:, :DW]
    mw = cw[:, DW:]
    scale = jnp.float32(1.0 / S)
    dn = (((1,), (1,)), ((), ()))
    acc = lax.dot_general(w[:, 0 * DW:1 * DW], _lo_f32(sw), dn,
                          preferred_element_type=jnp.float32)
    acc += lax.dot_general(w[:, 1 * DW:2 * DW], _hi_f32(sw), dn,
                           preferred_element_type=jnp.float32)
    acc += lax.dot_general(w[:, 2 * DW:3 * DW], _lo_f32(mw) * scale, dn,
                           preferred_element_type=jnp.float32)
    acc += lax.dot_general(w[:, 3 * DW:4 * DW], _hi_f32(mw) * scale, dn,
                           preferred_element_type=jnp.float32)
    o_ref[...] = jnp.maximum(acc, 0.0)


def kernel(nodes, neigh_idx, features, weight):
    batch = nodes.shape[0]
    pad = BP - batch
    nodes_p = jnp.concatenate(
        [nodes.astype(jnp.int32), jnp.zeros((pad,), jnp.int32)])
    neigh_p = jnp.concatenate(
        [neigh_idx.astype(jnp.int32).reshape(-1),
         jnp.zeros((pad * S,), jnp.int32)])

    n_nodes = features.shape[0]
    feat_packed = pl.pallas_call(
        _pack_body,
        grid=(n_nodes // PACK_ROWS,),
        in_specs=[pl.BlockSpec((PACK_ROWS, D), lambda i: (i, 0))],
        out_specs=pl.BlockSpec((PACK_ROWS, DW), lambda i: (i, 0)),
        out_shape=jax.ShapeDtypeStruct((n_nodes, DW), jnp.int32),
    )(features)

    comb_i32 = _sc_gather_sum(feat_packed, nodes_p, neigh_p)

    nsteps = B_TC // TCG
    nn = neigh_p[B_SC * S:].reshape(nsteps, TCG, S).transpose(0, 2, 1)
    sn = nodes_p[B_SC:].reshape(nsteps, 1, TCG)
    tcidx = jnp.concatenate([nn, sn], axis=1).reshape(-1)

    comb_f32 = pl.pallas_call(
        _tc_gather_body,
        grid_spec=pltpu.PrefetchScalarGridSpec(
            num_scalar_prefetch=1,
            grid=(nsteps,),
            in_specs=[pl.BlockSpec(memory_space=pl.ANY)],
            out_specs=pl.BlockSpec((TCG, 2 * D), lambda i, *_: (i, 0)),
            scratch_shapes=[
                pltpu.VMEM((TCROWS, D), jnp.float32),
                pltpu.VMEM((TCROWS, D), jnp.float32),
                pltpu.SemaphoreType.DMA,
                pltpu.SemaphoreType.DMA,
            ],
        ),
        out_shape=jax.ShapeDtypeStruct((B_TC, 2 * D), jnp.float32),
    )(tcidx, features)

    out_sc = pl.pallas_call(
        _mm_body,
        grid=(B_SC // TB,),
        in_specs=[
            pl.BlockSpec((EMB, 2 * D), lambda i: (0, 0)),
            pl.BlockSpec((TB, 2 * DW), lambda i: (i, 0)),
        ],
        out_specs=pl.BlockSpec((EMB, TB), lambda i: (0, i)),
        out_shape=jax.ShapeDtypeStruct((EMB, B_SC), jnp.float32),
    )(weight, comb_i32)

    out_tc = pl.pallas_call(
        _mm_f32_body,
        grid=(1,),
        in_specs=[
            pl.BlockSpec((EMB, 2 * D), lambda i: (0, 0)),
            pl.BlockSpec((B_TC, 2 * D), lambda i: (0, 0)),
        ],
        out_specs=pl.BlockSpec((EMB, B_TC), lambda i: (0, 0)),
        out_shape=jax.ShapeDtypeStruct((EMB, B_TC), jnp.float32),
    )(weight, comb_f32)

    return jnp.concatenate([out_sc, out_tc], axis=1)[:, :batch]


# speedup vs baseline: 1.1651x; 1.1651x over previous
import functools

import jax
import jax.numpy as jnp
from jax import lax
from jax.experimental import pallas as pl
from jax.experimental.pallas import tpu as pltpu
from jax.experimental.pallas import tpu_sc as plsc

D = 256
DW = D // 2
S = 16
EMB = 256
NC = 2
NS = 16
NW = NC * NS
BP = 10240
B_SC = 8960
B_TC = BP - B_SC
SEEDS_PER_W = B_SC // NW
CS = 8
NCHUNK = SEEDS_PER_W // CS
NBUF = 5
TB = 1792
PACK_ROWS = 2000
TCG = 32
TCROWS = TCG * (S + 1)


def _rne16(f):
    u = lax.bitcast_convert_type(f, jnp.int32)
    odd = lax.bitwise_and(
        lax.shift_right_logical(u, jnp.int32(16)), jnp.int32(1))
    r = lax.shift_right_logical(u + jnp.int32(32767) + odd, jnp.int32(16))
    return lax.bitwise_and(r, jnp.int32(65535))


def _lo_f32(x):
    return lax.bitcast_convert_type(lax.shift_left(x, jnp.int32(16)),
                                    jnp.float32)


def _hi_f32(x):
    return lax.bitcast_convert_type(lax.bitwise_and(x, jnp.int32(-65536)),
                                    jnp.float32)


def _pack_body(f_ref, o_ref):
    x = f_ref[...]
    lo = _rne16(x[:, :DW])
    hi = _rne16(x[:, DW:])
    o_ref[...] = lax.bitwise_or(lax.shift_left(hi, jnp.int32(16)), lo)


def _make_sc_gather_sum():
    mesh = plsc.VectorSubcoreMesh(core_axis_name="c", subcore_axis_name="s")

    @functools.partial(
        pl.kernel,
        mesh=mesh,
        out_type=jax.ShapeDtypeStruct((B_SC, 2 * DW), jnp.int32),
        scratch_types=(
            [pltpu.VMEM((SEEDS_PER_W * S,), jnp.int32),
             pltpu.VMEM((SEEDS_PER_W,), jnp.int32)]
            + [pltpu.VMEM((CS * S, DW), jnp.int32)] * NBUF
            + [pltpu.VMEM((CS, DW), jnp.int32)] * NBUF
            + [pltpu.VMEM((CS, 2 * DW), jnp.int32)] * NBUF
            + [pltpu.SemaphoreType.DMA] * (2 * NBUF)
        ),
    )
    def sc_gather_sum(feat_hbm, nodes_hbm, neigh_hbm, comb_out,
                      nidx_v, sidx_v, *bufs):
        nbufs = bufs[0:NBUF]
        sbufs = bufs[NBUF:2 * NBUF]
        obufs = bufs[2 * NBUF:3 * NBUF]
        gsems = bufs[3 * NBUF:4 * NBUF]
        osems = bufs[4 * NBUF:5 * NBUF]
        wid = lax.axis_index("s") * NC + lax.axis_index("c")
        base = pl.multiple_of(wid * SEEDS_PER_W, SEEDS_PER_W)
        pltpu.sync_copy(neigh_hbm.at[pl.ds(base * S, SEEDS_PER_W * S)], nidx_v)
        pltpu.sync_copy(nodes_hbm.at[pl.ds(base, SEEDS_PER_W)], sidx_v)

        def fire_gather(g, b):
            off_n = pl.multiple_of(g * (CS * S), CS * S)
            off_s = pl.multiple_of(g * CS, CS)
            pltpu.async_copy(
                feat_hbm.at[nidx_v.at[pl.ds(off_n, CS * S)]], nbufs[b], gsems[b])
            pltpu.async_copy(
                feat_hbm.at[sidx_v.at[pl.ds(off_s, CS)]], sbufs[b], gsems[b])

        def wait_gather(b):
            pltpu.make_async_copy(
                feat_hbm.at[pl.ds(0, CS * S)], nbufs[b], gsems[b]).wait()
            pltpu.make_async_copy(
                feat_hbm.at[pl.ds(0, CS)], sbufs[b], gsems[b]).wait()

        def fire_out(g, b):
            row = pl.multiple_of(base + g * CS, CS)
            pltpu.async_copy(obufs[b], comb_out.at[pl.ds(row, CS)], osems[b])

        def drain_out(b):
            pltpu.make_async_copy(
                obufs[b], comb_out.at[pl.ds(0, CS)], osems[b]).wait()

        def compute(b):
            nb, sb, ob = nbufs[b], sbufs[b], obufs[b]

            def seed_body(s0, _):
                r0 = s0 * S
                for v in range(DW // 16):
                    x = nb[r0, pl.ds(v * 16, 16)]
                    a_lo, a_hi = _lo_f32(x), _hi_f32(x)
                    for r in range(1, S):
                        y = nb[r0 + r, pl.ds(v * 16, 16)]
                        a_lo = a_lo + _lo_f32(y)
                        a_hi = a_hi + _hi_f32(y)
                    word = lax.bitwise_or(
                        lax.shift_left(_rne16(a_hi), jnp.int32(16)),
                        _rne16(a_lo))
                    ob[s0, pl.ds(DW + v * 16, 16)] = word
                    ob[s0, pl.ds(v * 16, 16)] = sb[s0, pl.ds(v * 16, 16)]
                return 0

            lax.fori_loop(0, CS, seed_body, 0, unroll=False)

        for b in range(NBUF):
            fire_gather(b, b)

        def ring_body(p, _):
            for b in range(NBUF):
                g = p * NBUF + b
                wait_gather(b)
                compute(b)

                @pl.when(p > 0)
                def _():
                    drain_out(b)

                fire_out(g, b)

                @pl.when(g + NBUF < NCHUNK)
                def _():
                    fire_gather(g + NBUF, b)
            return 0

        lax.fori_loop(0, NCHUNK // NBUF, ring_body, 0, unroll=False)
        for b in range(NBUF):
            drain_out(b)

    return sc_gather_sum


_sc_gather_sum = _make_sc_gather_sum()


def _tc_gather_body(idx_ref, feat_ref, comb_ref, rows0, rows1, sem0, sem1):
    i = pl.program_id(0)
    n = pl.num_programs(0)

    def fire(step, buf, sem):
        base = step * TCROWS
        for j in range(TCROWS):
            pltpu.make_async_copy(
                feat_ref.at[pl.ds(idx_ref[base + j], 1)],
                buf.at[pl.ds(j, 1)], sem).start()

    def drain(buf, sem):
        for j in range(TCROWS):
            pltpu.make_async_copy(
                feat_ref.at[pl.ds(0, 1)], buf.at[pl.ds(j, 1)], sem).wait()

    def consume(buf):
        x = buf[...]
        acc = x[0:TCG]
        for k in range(1, S):
            acc = acc + x[k * TCG:(k + 1) * TCG]
        sf = x[S * TCG:(S + 1) * TCG]
        comb_ref[...] = jnp.concatenate(
            [sf, acc * jnp.float32(1.0 / S)], axis=1)

    @pl.when(i == 0)
    def _():
        fire(0, rows0, sem0)

    @pl.when(i % 2 == 0)
    def _():
        @pl.when(i + 1 < n)
        def _():
            fire(i + 1, rows1, sem1)

        drain(rows0, sem0)
        consume(rows0)

    @pl.when(i % 2 == 1)
    def _():
        @pl.when(i + 1 < n)
        def _():
            fire(i + 1, rows0, sem0)

        drain(rows1, sem1)
        consume(rows1)


def _mm_f32_body(w_ref, c_ref, o_ref):
    w = w_ref[...]
    dn = (((1,), (1,)), ((), ()))
    acc = lax.dot_general(w[:, :D], c_ref[:, :D], dn,
                          preferred_element_type=jnp.float32)
    acc += lax.dot_general(w[:, D:], c_ref[:, D:], dn,
                           preferred_element_type=jnp.float32)
    o_ref[...] = jnp.maximum(acc, 0.0)


def _mm_body(w_ref, c_ref, o_ref):
    w = w_ref[...]
    cw = c_ref[...]
    sw = cw[:, :DW]
    mw = cw[:, DW:]
    scale = jnp.float32(1.0 / S)
    dn = (((1,), (1,)), ((), ()))
    acc = lax.dot_general(w[:, 0 * DW:1 * DW], _lo_f32(sw), dn,
                          preferred_element_type=jnp.float32)
    acc += lax.dot_general(w[:, 1 * DW:2 * DW], _hi_f32(sw), dn,
                           preferred_element_type=jnp.float32)
    acc += lax.dot_general(w[:, 2 * DW:3 * DW], _lo_f32(mw) * scale, dn,
                           preferred_element_type=jnp.float32)
    acc += lax.dot_general(w[:, 3 * DW:4 * DW], _hi_f32(mw) * scale, dn,
                           preferred_element_type=jnp.float32)
    o_ref[...] = jnp.maximum(acc, 0.0)


def kernel(nodes, neigh_idx, features, weight):
    batch = nodes.shape[0]
    pad = BP - batch
    nodes_p = jnp.concatenate(
        [nodes.astype(jnp.int32), jnp.zeros((pad,), jnp.int32)])
    neigh_p = jnp.concatenate(
        [neigh_idx.astype(jnp.int32).reshape(-1),
         jnp.zeros((pad * S,), jnp.int32)])

    n_nodes = features.shape[0]
    feat_packed = pl.pallas_call(
        _pack_body,
        grid=(n_nodes // PACK_ROWS,),
        in_specs=[pl.BlockSpec((PACK_ROWS, D), lambda i: (i, 0))],
        out_specs=pl.BlockSpec((PACK_ROWS, DW), lambda i: (i, 0)),
        out_shape=jax.ShapeDtypeStruct((n_nodes, DW), jnp.int32),
    )(features)

    comb_i32 = _sc_gather_sum(feat_packed, nodes_p, neigh_p)

    nsteps = B_TC // TCG
    nn = neigh_p[B_SC * S:].reshape(nsteps, TCG, S).transpose(0, 2, 1)
    sn = nodes_p[B_SC:].reshape(nsteps, 1, TCG)
    tcidx = jnp.concatenate([nn, sn], axis=1).reshape(-1)

    comb_f32 = pl.pallas_call(
        _tc_gather_body,
        grid_spec=pltpu.PrefetchScalarGridSpec(
            num_scalar_prefetch=1,
            grid=(nsteps,),
            in_specs=[pl.BlockSpec(memory_space=pl.ANY)],
            out_specs=pl.BlockSpec((TCG, 2 * D), lambda i, *_: (i, 0)),
            scratch_shapes=[
                pltpu.VMEM((TCROWS, D), jnp.float32),
                pltpu.VMEM((TCROWS, D), jnp.float32),
                pltpu.SemaphoreType.DMA,
                pltpu.SemaphoreType.DMA,
            ],
        ),
        out_shape=jax.ShapeDtypeStruct((B_TC, 2 * D), jnp.float32),
    )(tcidx, features)

    out_sc = pl.pallas_call(
        _mm_body,
        grid=(B_SC // TB,),
        in_specs=[
            pl.BlockSpec((EMB, 2 * D), lambda i: (0, 0)),
            pl.BlockSpec((TB, 2 * DW), lambda i: (i, 0)),
        ],
        out_specs=pl.BlockSpec((EMB, TB), lambda i: (0, i)),
        out_shape=jax.ShapeDtypeStruct((EMB, B_SC), jnp.float32),
    )(weight, comb_i32)

    out_tc = pl.pallas_call(
        _mm_f32_body,
        grid=(1,),
        in_specs=[
            pl.BlockSpec((EMB, 2 * D), lambda i: (0, 0)),
            pl.BlockSpec((B_TC, 2 * D), lambda i: (0, 0)),
        ],
        out_specs=pl.BlockSpec((EMB, B_TC), lambda i: (0, 0)),
        out_shape=jax.ShapeDtypeStruct((EMB, B_TC), jnp.float32),
    )(weight, comb_f32)

    return jnp.concatenate([out_sc, out_tc], axis=1)[:, :batch]


# speedup vs baseline: 1.2605x; 1.0818x over previous
import functools

import jax
import jax.numpy as jnp
from jax import lax
from jax.experimental import pallas as pl
from jax.experimental.pallas import tpu as pltpu
from jax.experimental.pallas import tpu_sc as plsc

D = 256
DW = D // 2
S = 16
EMB = 256
NC = 2
NS = 16
NW = NC * NS
BP = 10240
SEEDS_PER_W = BP // NW
CS = 8
NCHUNK = SEEDS_PER_W // CS
NBUF = 4
TB = 2048
PACK_ROWS = 2000


def _rne16(f):
    u = lax.bitcast_convert_type(f, jnp.int32)
    odd = lax.bitwise_and(
        lax.shift_right_logical(u, jnp.int32(16)), jnp.int32(1))
    r = lax.shift_right_logical(u + jnp.int32(32767) + odd, jnp.int32(16))
    return lax.bitwise_and(r, jnp.int32(65535))


def _lo_f32(x):
    return lax.bitcast_convert_type(lax.shift_left(x, jnp.int32(16)),
                                    jnp.float32)


def _hi_f32(x):
    return lax.bitcast_convert_type(lax.bitwise_and(x, jnp.int32(-65536)),
                                    jnp.float32)


def _pack_body(f_ref, o_ref):
    x = f_ref[...]
    lo = _rne16(x[:, :DW])
    hi = _rne16(x[:, DW:])
    o_ref[...] = lax.bitwise_or(lax.shift_left(hi, jnp.int32(16)), lo)


def _make_sc_gather_sum():
    mesh = plsc.VectorSubcoreMesh(core_axis_name="c", subcore_axis_name="s")

    @functools.partial(
        pl.kernel,
        mesh=mesh,
        out_type=jax.ShapeDtypeStruct((BP, 2 * DW), jnp.int32),
        scratch_types=(
            [pltpu.VMEM((SEEDS_PER_W * S,), jnp.int32),
             pltpu.VMEM((SEEDS_PER_W,), jnp.int32)]
            + [pltpu.VMEM((CS * S, DW), jnp.int32)] * NBUF
            + [pltpu.VMEM((CS, DW), jnp.int32)] * NBUF
            + [pltpu.VMEM((CS, 2 * DW), jnp.int32)] * NBUF
            + [pltpu.SemaphoreType.DMA] * (2 * NBUF)
        ),
    )
    def sc_gather_sum(feat_hbm, nodes_hbm, neigh_hbm, comb_out,
                      nidx_v, sidx_v, *bufs):
        nbufs = bufs[0:NBUF]
        sbufs = bufs[NBUF:2 * NBUF]
        obufs = bufs[2 * NBUF:3 * NBUF]
        gsems = bufs[3 * NBUF:4 * NBUF]
        osems = bufs[4 * NBUF:5 * NBUF]
        wid = lax.axis_index("s") * NC + lax.axis_index("c")
        base = pl.multiple_of(wid * SEEDS_PER_W, SEEDS_PER_W)
        pltpu.sync_copy(neigh_hbm.at[pl.ds(base * S, SEEDS_PER_W * S)], nidx_v)
        pltpu.sync_copy(nodes_hbm.at[pl.ds(base, SEEDS_PER_W)], sidx_v)

        def fire_gather(g, b):
            off_n = pl.multiple_of(g * (CS * S), CS * S)
            off_s = pl.multiple_of(g * CS, CS)
            pltpu.async_copy(
                feat_hbm.at[nidx_v.at[pl.ds(off_n, CS * S)]], nbufs[b], gsems[b])
            pltpu.async_copy(
                feat_hbm.at[sidx_v.at[pl.ds(off_s, CS)]], sbufs[b], gsems[b])

        def wait_gather(b):
            pltpu.make_async_copy(
                feat_hbm.at[pl.ds(0, CS * S)], nbufs[b], gsems[b]).wait()
            pltpu.make_async_copy(
                feat_hbm.at[pl.ds(0, CS)], sbufs[b], gsems[b]).wait()

        def fire_out(g, b):
            row = pl.multiple_of(base + g * CS, CS)
            pltpu.async_copy(obufs[b], comb_out.at[pl.ds(row, CS)], osems[b])

        def drain_out(b):
            pltpu.make_async_copy(
                obufs[b], comb_out.at[pl.ds(0, CS)], osems[b]).wait()

        def compute(b):
            nb, sb, ob = nbufs[b], sbufs[b], obufs[b]

            def seed_body(s0, _):
                r0 = s0 * S
                for v in range(DW // 16):
                    x = nb[r0, pl.ds(v * 16, 16)]
                    a_lo, a_hi = _lo_f32(x), _hi_f32(x)
                    for r in range(1, S):
                        y = nb[r0 + r, pl.ds(v * 16, 16)]
                        a_lo = a_lo + _lo_f32(y)
                        a_hi = a_hi + _hi_f32(y)
                    word = lax.bitwise_or(
                        lax.shift_left(_rne16(a_hi), jnp.int32(16)),
                        _rne16(a_lo))
                    ob[s0, pl.ds(DW + v * 16, 16)] = word
                    ob[s0, pl.ds(v * 16, 16)] = sb[s0, pl.ds(v * 16, 16)]
                return 0

            lax.fori_loop(0, CS, seed_body, 0, unroll=False)

        for b in range(NBUF):
            fire_gather(b, b)

        def ring_body(p, _):
            for b in range(NBUF):
                g = p * NBUF + b
                wait_gather(b)
                compute(b)

                @pl.when(p > 0)
                def _():
                    drain_out(b)

                fire_out(g, b)

                @pl.when(g + NBUF < NCHUNK)
                def _():
                    fire_gather(g + NBUF, b)
            return 0

        lax.fori_loop(0, NCHUNK // NBUF, ring_body, 0, unroll=False)
        for b in range(NBUF):
            drain_out(b)

    return sc_gather_sum


_sc_gather_sum = _make_sc_gather_sum()


def _mm_body(w_ref, c_ref, o_ref):
    w = w_ref[...]
    cw = c_ref[...]
    sw = cw[:, :DW]
    mw = cw[:, DW:]
    scale = jnp.float32(1.0 / S)
    dn = (((1,), (1,)), ((), ()))
    acc = lax.dot_general(w[:, 0 * DW:1 * DW], _lo_f32(sw), dn,
                          preferred_element_type=jnp.float32)
    acc += lax.dot_general(w[:, 1 * DW:2 * DW], _hi_f32(sw), dn,
                           preferred_element_type=jnp.float32)
    acc += lax.dot_general(w[:, 2 * DW:3 * DW], _lo_f32(mw) * scale, dn,
                           preferred_element_type=jnp.float32)
    acc += lax.dot_general(w[:, 3 * DW:4 * DW], _hi_f32(mw) * scale, dn,
                           preferred_element_type=jnp.float32)
    o_ref[...] = jnp.maximum(acc, 0.0)


def kernel(nodes, neigh_idx, features, weight):
    batch = nodes.shape[0]
    pad = BP - batch
    nodes_p = jnp.concatenate(
        [nodes.astype(jnp.int32), jnp.zeros((pad,), jnp.int32)])
    neigh_p = jnp.concatenate(
        [neigh_idx.astype(jnp.int32).reshape(-1),
         jnp.zeros((pad * S,), jnp.int32)])

    n_nodes = features.shape[0]
    feat_packed = pl.pallas_call(
        _pack_body,
        grid=(n_nodes // PACK_ROWS,),
        in_specs=[pl.BlockSpec((PACK_ROWS, D), lambda i: (i, 0))],
        out_specs=pl.BlockSpec((PACK_ROWS, DW), lambda i: (i, 0)),
        out_shape=jax.ShapeDtypeStruct((n_nodes, DW), jnp.int32),
    )(features)

    comb_i32 = _sc_gather_sum(feat_packed, nodes_p, neigh_p)

    out_full = pl.pallas_call(
        _mm_body,
        grid=(BP // TB,),
        in_specs=[
            pl.BlockSpec((EMB, 2 * D), lambda i: (0, 0)),
            pl.BlockSpec((TB, 2 * DW), lambda i: (i, 0)),
        ],
        out_specs=pl.BlockSpec((EMB, TB), lambda i: (0, i)),
        out_shape=jax.ShapeDtypeStruct((EMB, BP), jnp.float32),
    )(weight, comb_i32)
    return out_full[:, :batch]
